# Initial kernel scaffold; baseline (speedup 1.0000x reference)
import jax
import jax.numpy as jnp
from jax.experimental import pallas as pl


def kernel(z, out_edge_index, batch, mlp_w1, mlp_b1, mlp_w2, mlp_b2, efc1_w, efc1_b, efc2_w, efc2_b):
    raise NotImplementedError("write your pallas kernel here")


# speedup vs baseline: 1.4881x; 1.4881x over previous
import functools

import jax
import jax.numpy as jnp
from jax import lax
from jax.experimental import pallas as pl
from jax.experimental.pallas import tpu as pltpu
from jax.experimental.pallas import tpu_sc as plsc

N = 10000
E = 320000
D = 128
H2 = 64
EFD = 16

NC = 2
NS = 16
NW = NC * NS
EP = 327680
EPW = EP // NW
CHUNK = 256
NCHUNK = EPW // CHUNK
IDXROWS = CHUNK // 128


def _pre_body(z_ref, wr_ref, wc_ref, bias_ref, a_ref, b_ref):
    zb = z_ref[...]
    s = jnp.sum(zb * zb, axis=1, keepdims=True)
    zn = zb / jnp.sqrt(s)
    a_ref[...] = (
        jnp.dot(zn, wr_ref[...], preferred_element_type=jnp.float32)
        + bias_ref[...]
    )
    b_ref[...] = jnp.dot(zn, wc_ref[...], preferred_element_type=jnp.float32)


def _tc_pre(z, wr, wc, bias):
    blk = 1000
    grid = N // blk
    return pl.pallas_call(
        _pre_body,
        grid=(grid,),
        in_specs=[
            pl.BlockSpec((blk, D), lambda i: (i, 0)),
            pl.BlockSpec((D, D), lambda i: (0, 0)),
            pl.BlockSpec((D, D), lambda i: (0, 0)),
            pl.BlockSpec((1, D), lambda i: (0, 0)),
        ],
        out_specs=[
            pl.BlockSpec((blk, D), lambda i: (i, 0)),
            pl.BlockSpec((blk, D), lambda i: (i, 0)),
        ],
        out_shape=[
            jax.ShapeDtypeStruct((N, D), jnp.float32),
            jax.ShapeDtypeStruct((N, D), jnp.float32),
        ],
    )(z, wr, wc, bias)


def _sc_body(a_hbm, b_hbm, row_hbm, col_hbm, out_hbm,
             ridx, cidx, buf_a, buf_b, sem_a, sem_b):
    wid = lax.axis_index("s") * NC + lax.axis_index("c")

    def chunk_body(i, carry):
        ebase = wid * EPW + i * CHUNK
        ibase = wid * (EPW // 128) + i * IDXROWS
        pltpu.sync_copy(row_hbm.at[pl.ds(ibase, IDXROWS)], ridx)
        pltpu.sync_copy(col_hbm.at[pl.ds(ibase, IDXROWS)], cidx)
        copies = []
        for j in range(IDXROWS):
            copies.append(pltpu.async_copy(
                a_hbm.at[ridx.at[j]], buf_a.at[pl.ds(j * 128, 128)], sem_a))
            copies.append(pltpu.async_copy(
                b_hbm.at[cidx.at[j]], buf_b.at[pl.ds(j * 128, 128)], sem_b))
        for c in copies:
            c.wait()

        def row_body(r, carry2):
            for k in range(D // 16):
                va = buf_a[r, pl.ds(k * 16, 16)]
                vb = buf_b[r, pl.ds(k * 16, 16)]
                buf_a[r, pl.ds(k * 16, 16)] = jnp.maximum(va + vb, 0.0)
            return carry2

        lax.fori_loop(0, CHUNK, row_body, 0)
        pltpu.sync_copy(buf_a, out_hbm.at[pl.ds(ebase, CHUNK)])
        return carry

    lax.fori_loop(0, NCHUNK, chunk_body, 0)


def _sc_gather_relu(a, b, row2d, col2d):
    mesh = plsc.VectorSubcoreMesh(core_axis_name="c", subcore_axis_name="s")
    return pl.kernel(
        _sc_body,
        out_type=jax.ShapeDtypeStruct((EP, D), jnp.float32),
        mesh=mesh,
        scratch_types=[
            pltpu.VMEM((IDXROWS, 128), jnp.int32),
            pltpu.VMEM((IDXROWS, 128), jnp.int32),
            pltpu.VMEM((CHUNK, D), jnp.float32),
            pltpu.VMEM((CHUNK, D), jnp.float32),
            pltpu.SemaphoreType.DMA,
            pltpu.SemaphoreType.DMA,
        ],
    )(a, b, row2d, col2d)


def _post_body(h_ref, w2_ref, b2_ref, w3_ref, b3_ref, prob_ref, attr_ref):
    h = h_ref[...]
    hm = h[:, :H2]
    ha = h[:, H2:]
    logit = jnp.sum(hm * w2_ref[...], axis=1, keepdims=True) + b2_ref[0, 0]
    prob_ref[...] = jax.nn.sigmoid(logit)
    att = (
        jnp.dot(ha, w3_ref[...], preferred_element_type=jnp.float32)
        + b3_ref[...]
    )
    att = att - jnp.max(att, axis=1, keepdims=True)
    ex = jnp.exp(att)
    attr_ref[...] = ex / jnp.sum(ex, axis=1, keepdims=True)


def _tc_post(h, w2row, b2, w3, b3):
    blk = 2560
    grid = E // blk
    return pl.pallas_call(
        _post_body,
        grid=(grid,),
        in_specs=[
            pl.BlockSpec((blk, D), lambda i: (i, 0)),
            pl.BlockSpec((1, H2), lambda i: (0, 0)),
            pl.BlockSpec((1, 1), lambda i: (0, 0)),
            pl.BlockSpec((H2, EFD), lambda i: (0, 0)),
            pl.BlockSpec((1, EFD), lambda i: (0, 0)),
        ],
        out_specs=[
            pl.BlockSpec((blk, 1), lambda i: (i, 0)),
            pl.BlockSpec((blk, EFD), lambda i: (i, 0)),
        ],
        out_shape=[
            jax.ShapeDtypeStruct((E, 1), jnp.float32),
            jax.ShapeDtypeStruct((E, EFD), jnp.float32),
        ],
    )(h, w2row, b2, w3, b3)


def kernel(z, out_edge_index, batch, mlp_w1, mlp_b1, mlp_w2, mlp_b2,
           efc1_w, efc1_b, efc2_w, efc2_b):
    wr = jnp.concatenate([mlp_w1[:D], efc1_w[:D]], axis=1)
    wc = jnp.concatenate([mlp_w1[D:], efc1_w[D:]], axis=1)
    bias = jnp.concatenate([mlp_b1, efc1_b]).reshape(1, 2 * H2)

    a, b = _tc_pre(z, wr, wc, bias)

    row = out_edge_index[0]
    col = out_edge_index[1]
    pad = EP - E
    row2d = jnp.concatenate([row, jnp.zeros((pad,), jnp.int32)]).reshape(-1, 128)
    col2d = jnp.concatenate([col, jnp.zeros((pad,), jnp.int32)]).reshape(-1, 128)

    h = _sc_gather_relu(a, b, row2d, col2d)

    prob, attr = _tc_post(
        h,
        mlp_w2.reshape(1, H2),
        mlp_b2.reshape(1, 1),
        efc2_w,
        efc2_b.reshape(1, EFD),
    )
    return (prob.reshape(-1), attr)


# speedup vs baseline: 1.7120x; 1.1504x over previous
import functools

import jax
import jax.numpy as jnp
from jax import lax
from jax.experimental import pallas as pl
from jax.experimental.pallas import tpu as pltpu
from jax.experimental.pallas import tpu_sc as plsc

N = 10000
E = 320000
D = 128
H2 = 64
EFD = 16

NC = 2
NS = 16
NW = NC * NS
EP = 327680
EPW = EP // NW
CHUNK = 128
NCHUNK = EPW // CHUNK
NBUF = 2


def _pre_body(z_ref, wr_ref, wc_ref, bias_ref, a_ref, b_ref):
    zb = z_ref[...]
    s = jnp.sum(zb * zb, axis=1, keepdims=True)
    zn = zb / jnp.sqrt(s)
    a_ref[...] = (
        jnp.dot(zn, wr_ref[...], preferred_element_type=jnp.float32)
        + bias_ref[...]
    )
    b_ref[...] = jnp.dot(zn, wc_ref[...], preferred_element_type=jnp.float32)


def _tc_pre(z, wr, wc, bias):
    blk = 1000
    grid = N // blk
    return pl.pallas_call(
        _pre_body,
        grid=(grid,),
        in_specs=[
            pl.BlockSpec((blk, D), lambda i: (i, 0)),
            pl.BlockSpec((D, D), lambda i: (0, 0)),
            pl.BlockSpec((D, D), lambda i: (0, 0)),
            pl.BlockSpec((1, D), lambda i: (0, 0)),
        ],
        out_specs=[
            pl.BlockSpec((blk, D), lambda i: (i, 0)),
            pl.BlockSpec((blk, D), lambda i: (i, 0)),
        ],
        out_shape=[
            jax.ShapeDtypeStruct((N, D), jnp.float32),
            jax.ShapeDtypeStruct((N, D), jnp.float32),
        ],
    )(z, wr, wc, bias)


def _sc_body(a_hbm, b_hbm, row_hbm, col_hbm, out_hbm,
             ridx, cidx, ba0, ba1, bb0, bb1, hb0, hb1,
             sg0, sg1, sw0, sw1):
    wid = lax.axis_index("s") * NC + lax.axis_index("c")
    ba = (ba0, ba1)
    bb = (bb0, bb1)
    hb = (hb0, hb1)
    sg = (sg0, sg1)
    sw = (sw0, sw1)
    ebase = wid * EPW

    pltpu.sync_copy(row_hbm.at[pl.ds(wid * NCHUNK, NCHUNK)], ridx)
    pltpu.sync_copy(col_hbm.at[pl.ds(wid * NCHUNK, NCHUNK)], cidx)

    pltpu.async_copy(a_hbm.at[ridx.at[0]], ba[0], sg[0])
    pltpu.async_copy(b_hbm.at[cidx.at[0]], bb[0], sg[0])

    def outer(t, carry):
        for b in range(NBUF):
            c = t * NBUF + b
            s = b
            sn = (b + 1) % NBUF
            cn = lax.rem(c + 1, NCHUNK)
            pltpu.async_copy(a_hbm.at[ridx.at[cn]], ba[sn], sg[sn])
            pltpu.async_copy(b_hbm.at[cidx.at[cn]], bb[sn], sg[sn])
            pltpu.make_async_copy(a_hbm.at[pl.ds(0, CHUNK)], ba[s], sg[s]).wait()
            pltpu.make_async_copy(b_hbm.at[pl.ds(0, CHUNK)], bb[s], sg[s]).wait()

            @pl.when(c >= NBUF)
            def _():
                pltpu.make_async_copy(
                    hb[s], out_hbm.at[pl.ds(0, CHUNK)], sw[s]).wait()

            @plsc.parallel_loop(0, CHUNK, unroll=4)
            def _(r):
                for k in range(D // 16):
                    va = ba[s][r, pl.ds(k * 16, 16)]
                    vb = bb[s][r, pl.ds(k * 16, 16)]
                    hb[s][r, pl.ds(k * 16, 16)] = jnp.maximum(va + vb, 0.0)

            pltpu.async_copy(
                hb[s], out_hbm.at[pl.ds(ebase + c * CHUNK, CHUNK)], sw[s])
        return carry

    lax.fori_loop(0, NCHUNK // NBUF, outer, 0)

    pltpu.make_async_copy(a_hbm.at[pl.ds(0, CHUNK)], ba[0], sg[0]).wait()
    pltpu.make_async_copy(b_hbm.at[pl.ds(0, CHUNK)], bb[0], sg[0]).wait()
    for s in range(NBUF):
        pltpu.make_async_copy(hb[s], out_hbm.at[pl.ds(0, CHUNK)], sw[s]).wait()


def _sc_gather_relu(a, b, row2d, col2d):
    mesh = plsc.VectorSubcoreMesh(core_axis_name="c", subcore_axis_name="s")
    return pl.kernel(
        _sc_body,
        out_type=jax.ShapeDtypeStruct((EP, D), jnp.float32),
        mesh=mesh,
        scratch_types=[
            pltpu.VMEM((NCHUNK, 128), jnp.int32),
            pltpu.VMEM((NCHUNK, 128), jnp.int32),
            pltpu.VMEM((CHUNK, D), jnp.float32),
            pltpu.VMEM((CHUNK, D), jnp.float32),
            pltpu.VMEM((CHUNK, D), jnp.float32),
            pltpu.VMEM((CHUNK, D), jnp.float32),
            pltpu.VMEM((CHUNK, D), jnp.float32),
            pltpu.VMEM((CHUNK, D), jnp.float32),
            pltpu.SemaphoreType.DMA,
            pltpu.SemaphoreType.DMA,
            pltpu.SemaphoreType.DMA,
            pltpu.SemaphoreType.DMA,
        ],
    )(a, b, row2d, col2d)


def _post_body(h_ref, w2_ref, b2_ref, w3_ref, b3_ref, prob_ref, attr_ref):
    h = h_ref[...]
    hm = h[:, :H2]
    ha = h[:, H2:]
    logit = jnp.sum(hm * w2_ref[...], axis=1, keepdims=True) + b2_ref[0, 0]
    prob_ref[...] = jax.nn.sigmoid(logit)
    att = (
        jnp.dot(ha, w3_ref[...], preferred_element_type=jnp.float32)
        + b3_ref[...]
    )
    att = att - jnp.max(att, axis=1, keepdims=True)
    ex = jnp.exp(att)
    attr_ref[...] = ex / jnp.sum(ex, axis=1, keepdims=True)


def _tc_post(h, w2row, b2, w3, b3):
    blk = 2560
    grid = E // blk
    return pl.pallas_call(
        _post_body,
        grid=(grid,),
        in_specs=[
            pl.BlockSpec((blk, D), lambda i: (i, 0)),
            pl.BlockSpec((1, H2), lambda i: (0, 0)),
            pl.BlockSpec((1, 1), lambda i: (0, 0)),
            pl.BlockSpec((H2, EFD), lambda i: (0, 0)),
            pl.BlockSpec((1, EFD), lambda i: (0, 0)),
        ],
        out_specs=[
            pl.BlockSpec((blk, 1), lambda i: (i, 0)),
            pl.BlockSpec((blk, EFD), lambda i: (i, 0)),
        ],
        out_shape=[
            jax.ShapeDtypeStruct((E, 1), jnp.float32),
            jax.ShapeDtypeStruct((E, EFD), jnp.float32),
        ],
    )(h, w2row, b2, w3, b3)


def kernel(z, out_edge_index, batch, mlp_w1, mlp_b1, mlp_w2, mlp_b2,
           efc1_w, efc1_b, efc2_w, efc2_b):
    wr = jnp.concatenate([mlp_w1[:D], efc1_w[:D]], axis=1)
    wc = jnp.concatenate([mlp_w1[D:], efc1_w[D:]], axis=1)
    bias = jnp.concatenate([mlp_b1, efc1_b]).reshape(1, 2 * H2)

    a, b = _tc_pre(z, wr, wc, bias)

    row = out_edge_index[0]
    col = out_edge_index[1]
    pad = EP - E
    row2d = jnp.concatenate([row, jnp.zeros((pad,), jnp.int32)]).reshape(-1, 128)
    col2d = jnp.concatenate([col, jnp.zeros((pad,), jnp.int32)]).reshape(-1, 128)

    h = _sc_gather_relu(a, b, row2d, col2d)

    prob, attr = _tc_post(
        h,
        mlp_w2.reshape(1, H2),
        mlp_b2.reshape(1, 1),
        efc2_w,
        efc2_b.reshape(1, EFD),
    )
    return (prob.reshape(-1), attr)


# speedup vs baseline: 2.3461x; 1.3704x over previous
import functools

import jax
import jax.numpy as jnp
from jax import lax
from jax.experimental import pallas as pl
from jax.experimental.pallas import tpu as pltpu
from jax.experimental.pallas import tpu_sc as plsc

N = 10000
E = 320000
D = 128
H2 = 64
EFD = 16

NC = 2
NS = 16
NW = NC * NS
EP = 327680
EPW = EP // NW
CHUNK = 128
NCHUNK = EPW // CHUNK
NBUF = 2


def _pre_body(z_ref, wr_ref, wc_ref, bias_ref, a_ref, b_ref):
    zb = z_ref[...]
    s = jnp.sum(zb * zb, axis=1, keepdims=True)
    zn = zb / jnp.sqrt(s)
    a_ref[...] = (
        jnp.dot(zn, wr_ref[...], preferred_element_type=jnp.float32)
        + bias_ref[...]
    )
    b_ref[...] = jnp.dot(zn, wc_ref[...], preferred_element_type=jnp.float32)


def _tc_pre(z, wr, wc, bias):
    blk = 1000
    grid = N // blk
    return pl.pallas_call(
        _pre_body,
        grid=(grid,),
        in_specs=[
            pl.BlockSpec((blk, D), lambda i: (i, 0)),
            pl.BlockSpec((D, D), lambda i: (0, 0)),
            pl.BlockSpec((D, D), lambda i: (0, 0)),
            pl.BlockSpec((1, D), lambda i: (0, 0)),
        ],
        out_specs=[
            pl.BlockSpec((blk, D), lambda i: (i, 0)),
            pl.BlockSpec((blk, D), lambda i: (i, 0)),
        ],
        out_shape=[
            jax.ShapeDtypeStruct((N, D), jnp.float32),
            jax.ShapeDtypeStruct((N, D), jnp.float32),
        ],
    )(z, wr, wc, bias)


def _sc_body(a_hbm, b_hbm, row_hbm, col_hbm, out_hbm,
             ridx, cidx, ba0, ba1, bb0, bb1, hb0, hb1,
             sg0, sg1, sw0, sw1):
    wid = lax.axis_index("s") * NC + lax.axis_index("c")
    ba = (ba0, ba1)
    bb = (bb0, bb1)
    hb = (hb0, hb1)
    sg = (sg0, sg1)
    sw = (sw0, sw1)
    ebase = wid * EPW

    pltpu.sync_copy(row_hbm.at[pl.ds(wid * NCHUNK, NCHUNK)], ridx)
    pltpu.sync_copy(col_hbm.at[pl.ds(wid * NCHUNK, NCHUNK)], cidx)

    pltpu.async_copy(a_hbm.at[ridx.at[0]], ba[0], sg[0])
    pltpu.async_copy(b_hbm.at[cidx.at[0]], bb[0], sg[0])

    def outer(t, carry):
        for b in range(NBUF):
            c = t * NBUF + b
            s = b
            sn = (b + 1) % NBUF
            cn = lax.rem(c + 1, NCHUNK)
            pltpu.async_copy(a_hbm.at[ridx.at[cn]], ba[sn], sg[sn])
            pltpu.async_copy(b_hbm.at[cidx.at[cn]], bb[sn], sg[sn])
            pltpu.make_async_copy(a_hbm.at[pl.ds(0, CHUNK)], ba[s], sg[s]).wait()
            pltpu.make_async_copy(b_hbm.at[pl.ds(0, CHUNK)], bb[s], sg[s]).wait()

            @pl.when(c >= NBUF)
            def _():
                pltpu.make_async_copy(
                    hb[s], out_hbm.at[pl.ds(0, CHUNK)], sw[s]).wait()

            @plsc.parallel_loop(0, CHUNK, unroll=4)
            def _(r):
                for k in range(D // 16):
                    va = ba[s][r, pl.ds(k * 16, 16)]
                    vb = bb[s][r, pl.ds(k * 16, 16)]
                    hb[s][r, pl.ds(k * 16, 16)] = jnp.maximum(va + vb, 0.0)

            pltpu.async_copy(
                hb[s], out_hbm.at[pl.ds(ebase + c * CHUNK, CHUNK)], sw[s])
        return carry

    lax.fori_loop(0, NCHUNK // NBUF, outer, 0)

    pltpu.make_async_copy(a_hbm.at[pl.ds(0, CHUNK)], ba[0], sg[0]).wait()
    pltpu.make_async_copy(b_hbm.at[pl.ds(0, CHUNK)], bb[0], sg[0]).wait()
    for s in range(NBUF):
        pltpu.make_async_copy(hb[s], out_hbm.at[pl.ds(0, CHUNK)], sw[s]).wait()


def _sc_gather_relu(a, b, row2d, col2d):
    mesh = plsc.VectorSubcoreMesh(core_axis_name="c", subcore_axis_name="s")
    return pl.kernel(
        _sc_body,
        out_type=jax.ShapeDtypeStruct((EP, D), jnp.float32),
        mesh=mesh,
        scratch_types=[
            pltpu.VMEM((NCHUNK, 128), jnp.int32),
            pltpu.VMEM((NCHUNK, 128), jnp.int32),
            pltpu.VMEM((CHUNK, D), jnp.float32),
            pltpu.VMEM((CHUNK, D), jnp.float32),
            pltpu.VMEM((CHUNK, D), jnp.float32),
            pltpu.VMEM((CHUNK, D), jnp.float32),
            pltpu.VMEM((CHUNK, D), jnp.float32),
            pltpu.VMEM((CHUNK, D), jnp.float32),
            pltpu.SemaphoreType.DMA,
            pltpu.SemaphoreType.DMA,
            pltpu.SemaphoreType.DMA,
            pltpu.SemaphoreType.DMA,
        ],
    )(a, b, row2d, col2d)


def _post_body(h_ref, wt_ref, bt_ref, prob_ref, attr_ref):
    h = h_ref[...]
    y_t = jax.lax.dot_general(
        wt_ref[...], h, (((1,), (1,)), ((), ())),
        preferred_element_type=jnp.float32,
    ) + bt_ref[...]
    prob_ref[...] = jax.nn.sigmoid(y_t[0:1, :])
    att = y_t[1:, :]
    att = att - jnp.max(att, axis=0, keepdims=True)
    ex = jnp.exp(att)
    attr_ref[...] = ex / jnp.sum(ex, axis=0, keepdims=True)


def _tc_post(h, wt, bt):
    blk = 2560
    grid = E // blk
    return pl.pallas_call(
        _post_body,
        grid=(grid,),
        in_specs=[
            pl.BlockSpec((blk, D), lambda i: (i, 0)),
            pl.BlockSpec((EFD + 1, D), lambda i: (0, 0)),
            pl.BlockSpec((EFD + 1, 1), lambda i: (0, 0)),
        ],
        out_specs=[
            pl.BlockSpec((1, blk), lambda i: (0, i)),
            pl.BlockSpec((EFD, blk), lambda i: (0, i)),
        ],
        out_shape=[
            jax.ShapeDtypeStruct((1, E), jnp.float32),
            jax.ShapeDtypeStruct((EFD, E), jnp.float32),
        ],
    )(h, wt, bt)


def kernel(z, out_edge_index, batch, mlp_w1, mlp_b1, mlp_w2, mlp_b2,
           efc1_w, efc1_b, efc2_w, efc2_b):
    wr = jnp.concatenate([mlp_w1[:D], efc1_w[:D]], axis=1)
    wc = jnp.concatenate([mlp_w1[D:], efc1_w[D:]], axis=1)
    bias = jnp.concatenate([mlp_b1, efc1_b]).reshape(1, 2 * H2)

    a, b = _tc_pre(z, wr, wc, bias)

    row = out_edge_index[0]
    col = out_edge_index[1]
    pad = EP - E
    row2d = jnp.concatenate([row, jnp.zeros((pad,), jnp.int32)]).reshape(-1, 128)
    col2d = jnp.concatenate([col, jnp.zeros((pad,), jnp.int32)]).reshape(-1, 128)

    h = _sc_gather_relu(a, b, row2d, col2d)

    wt = jnp.concatenate([
        jnp.concatenate([mlp_w2.reshape(1, H2),
                         jnp.zeros((1, H2), jnp.float32)], axis=1),
        jnp.concatenate([jnp.zeros((EFD, H2), jnp.float32),
                         efc2_w.T], axis=1),
    ], axis=0)
    bt = jnp.concatenate([mlp_b2, efc2_b]).reshape(EFD + 1, 1)

    prob2d, attr_t = _tc_post(h, wt, bt)
    return (prob2d.reshape(-1), attr_t.T)


# speedup vs baseline: 2.4323x; 1.0367x over previous
import functools

import jax
import jax.numpy as jnp
from jax import lax
from jax.experimental import pallas as pl
from jax.experimental.pallas import tpu as pltpu
from jax.experimental.pallas import tpu_sc as plsc

N = 10000
E = 320000
D = 128
H2 = 64
EFD = 16

NC = 2
NS = 16
NW = NC * NS
EP = 327680
SLICES = 2
EPS = EP // SLICES
EPW = EPS // NW
CHUNK = 128
NCHUNK = EPW // CHUNK
NBUF = 2


def _pre_body(z_ref, wr_ref, wc_ref, bias_ref, a_ref, b_ref):
    zb = z_ref[...]
    s = jnp.sum(zb * zb, axis=1, keepdims=True)
    zn = zb / jnp.sqrt(s)
    a_ref[...] = (
        jnp.dot(zn, wr_ref[...], preferred_element_type=jnp.float32)
        + bias_ref[...]
    )
    b_ref[...] = jnp.dot(zn, wc_ref[...], preferred_element_type=jnp.float32)


def _tc_pre(z, wr, wc, bias):
    blk = 1000
    grid = N // blk
    return pl.pallas_call(
        _pre_body,
        grid=(grid,),
        in_specs=[
            pl.BlockSpec((blk, D), lambda i: (i, 0)),
            pl.BlockSpec((D, D), lambda i: (0, 0)),
            pl.BlockSpec((D, D), lambda i: (0, 0)),
            pl.BlockSpec((1, D), lambda i: (0, 0)),
        ],
        out_specs=[
            pl.BlockSpec((blk, D), lambda i: (i, 0)),
            pl.BlockSpec((blk, D), lambda i: (i, 0)),
        ],
        out_shape=[
            jax.ShapeDtypeStruct((N, D), jnp.float32),
            jax.ShapeDtypeStruct((N, D), jnp.float32),
        ],
    )(z, wr, wc, bias)


def _sc_body(a_hbm, b_hbm, row_hbm, col_hbm, out_hbm,
             ridx, cidx, ba0, ba1, bb0, bb1, hb0, hb1,
             sg0, sg1, sw0, sw1):
    wid = lax.axis_index("s") * NC + lax.axis_index("c")
    ba = (ba0, ba1)
    bb = (bb0, bb1)
    hb = (hb0, hb1)
    sg = (sg0, sg1)
    sw = (sw0, sw1)
    ebase = wid * EPW

    pltpu.sync_copy(row_hbm.at[pl.ds(wid * NCHUNK, NCHUNK)], ridx)
    pltpu.sync_copy(col_hbm.at[pl.ds(wid * NCHUNK, NCHUNK)], cidx)

    pltpu.async_copy(a_hbm.at[ridx.at[0]], ba[0], sg[0])
    pltpu.async_copy(b_hbm.at[cidx.at[0]], bb[0], sg[0])

    def outer(t, carry):
        for b in range(NBUF):
            c = t * NBUF + b
            s = b
            sn = (b + 1) % NBUF
            cn = lax.rem(c + 1, NCHUNK)
            pltpu.async_copy(a_hbm.at[ridx.at[cn]], ba[sn], sg[sn])
            pltpu.async_copy(b_hbm.at[cidx.at[cn]], bb[sn], sg[sn])
            pltpu.make_async_copy(a_hbm.at[pl.ds(0, CHUNK)], ba[s], sg[s]).wait()
            pltpu.make_async_copy(b_hbm.at[pl.ds(0, CHUNK)], bb[s], sg[s]).wait()

            @pl.when(c >= NBUF)
            def _():
                pltpu.make_async_copy(
                    hb[s], out_hbm.at[pl.ds(0, CHUNK)], sw[s]).wait()

            @plsc.parallel_loop(0, CHUNK, unroll=4)
            def _(r):
                for k in range(D // 32):
                    x0 = jnp.maximum(
                        ba[s][r, pl.ds(k * 32, 16)]
                        + bb[s][r, pl.ds(k * 32, 16)], 0.0)
                    x1 = jnp.maximum(
                        ba[s][r, pl.ds(k * 32 + 16, 16)]
                        + bb[s][r, pl.ds(k * 32 + 16, 16)], 0.0)
                    u0 = lax.bitcast_convert_type(x0, jnp.int32)
                    u1 = lax.bitcast_convert_type(x1, jnp.int32)
                    r0 = u0 + 0x7FFF + (lax.shift_right_logical(u0, 16) & 1)
                    r1 = u1 + 0x7FFF + (lax.shift_right_logical(u1, 16) & 1)
                    packed = lax.shift_right_logical(r0, 16) | (r1 & -65536)
                    hb[s][r, pl.ds(k * 16, 16)] = packed

            pltpu.async_copy(
                hb[s], out_hbm.at[pl.ds(ebase + c * CHUNK, CHUNK)], sw[s])
        return carry

    lax.fori_loop(0, NCHUNK // NBUF, outer, 0)

    pltpu.make_async_copy(a_hbm.at[pl.ds(0, CHUNK)], ba[0], sg[0]).wait()
    pltpu.make_async_copy(b_hbm.at[pl.ds(0, CHUNK)], bb[0], sg[0]).wait()
    for s in range(NBUF):
        pltpu.make_async_copy(hb[s], out_hbm.at[pl.ds(0, CHUNK)], sw[s]).wait()


def _sc_gather_relu(a, b, row2d, col2d):
    mesh = plsc.VectorSubcoreMesh(core_axis_name="c", subcore_axis_name="s")
    return pl.kernel(
        _sc_body,
        out_type=jax.ShapeDtypeStruct((EPS, D // 2), jnp.int32),
        mesh=mesh,
        scratch_types=[
            pltpu.VMEM((NCHUNK, 128), jnp.int32),
            pltpu.VMEM((NCHUNK, 128), jnp.int32),
            pltpu.VMEM((CHUNK, D), jnp.float32),
            pltpu.VMEM((CHUNK, D), jnp.float32),
            pltpu.VMEM((CHUNK, D), jnp.float32),
            pltpu.VMEM((CHUNK, D), jnp.float32),
            pltpu.VMEM((CHUNK, D // 2), jnp.int32),
            pltpu.VMEM((CHUNK, D // 2), jnp.int32),
            pltpu.SemaphoreType.DMA,
            pltpu.SemaphoreType.DMA,
            pltpu.SemaphoreType.DMA,
            pltpu.SemaphoreType.DMA,
        ],
    )(a, b, row2d, col2d)


def _post_body(h_ref, wtl_ref, wth_ref, bt_ref, prob_ref, attr_ref):
    hw = h_ref[...]
    lo = lax.bitcast_convert_type(lax.shift_left(hw, 16), jnp.float32)
    hi = lax.bitcast_convert_type(hw & -65536, jnp.float32)
    y_t = (
        jax.lax.dot_general(
            wtl_ref[...], lo, (((1,), (1,)), ((), ())),
            preferred_element_type=jnp.float32)
        + jax.lax.dot_general(
            wth_ref[...], hi, (((1,), (1,)), ((), ())),
            preferred_element_type=jnp.float32)
        + bt_ref[...]
    )
    prob_ref[...] = jax.nn.sigmoid(y_t[0:1, :])
    att = y_t[1:, :]
    att = att - jnp.max(att, axis=0, keepdims=True)
    ex = jnp.exp(att)
    attr_ref[...] = ex / jnp.sum(ex, axis=0, keepdims=True)


def _tc_post(h, wtl, wth, bt, ne):
    blk = 2560
    grid = ne // blk
    return pl.pallas_call(
        _post_body,
        grid=(grid,),
        in_specs=[
            pl.BlockSpec((blk, D // 2), lambda i: (i, 0)),
            pl.BlockSpec((EFD + 1, D // 2), lambda i: (0, 0)),
            pl.BlockSpec((EFD + 1, D // 2), lambda i: (0, 0)),
            pl.BlockSpec((EFD + 1, 1), lambda i: (0, 0)),
        ],
        out_specs=[
            pl.BlockSpec((1, blk), lambda i: (0, i)),
            pl.BlockSpec((EFD, blk), lambda i: (0, i)),
        ],
        out_shape=[
            jax.ShapeDtypeStruct((1, ne), jnp.float32),
            jax.ShapeDtypeStruct((EFD, ne), jnp.float32),
        ],
    )(h, wtl, wth, bt)


def kernel(z, out_edge_index, batch, mlp_w1, mlp_b1, mlp_w2, mlp_b2,
           efc1_w, efc1_b, efc2_w, efc2_b):
    wr = jnp.concatenate([mlp_w1[:D], efc1_w[:D]], axis=1)
    wc = jnp.concatenate([mlp_w1[D:], efc1_w[D:]], axis=1)
    bias = jnp.concatenate([mlp_b1, efc1_b]).reshape(1, 2 * H2)

    a, b = _tc_pre(z, wr, wc, bias)

    row = out_edge_index[0]
    col = out_edge_index[1]
    pad = EP - E
    row2d = jnp.concatenate([row, jnp.zeros((pad,), jnp.int32)]).reshape(-1, 128)
    col2d = jnp.concatenate([col, jnp.zeros((pad,), jnp.int32)]).reshape(-1, 128)

    hs = [
        _sc_gather_relu(a, b,
                        row2d[i * (EPS // 128):(i + 1) * (EPS // 128)],
                        col2d[i * (EPS // 128):(i + 1) * (EPS // 128)])
        for i in range(SLICES)
    ]

    wt = jnp.concatenate([
        jnp.concatenate([mlp_w2.reshape(1, H2),
                         jnp.zeros((1, H2), jnp.float32)], axis=1),
        jnp.concatenate([jnp.zeros((EFD, H2), jnp.float32),
                         efc2_w.T], axis=1),
    ], axis=0)
    bt = jnp.concatenate([mlp_b2, efc2_b]).reshape(EFD + 1, 1)
    qlo = jnp.array([32 * (q // 16) + q % 16 for q in range(D // 2)], jnp.int32)
    wtl = wt[:, qlo]
    wth = wt[:, qlo + 16]

    outs = []
    for i in range(SLICES):
        ne = min(E - i * EPS, EPS)
        outs.append(_tc_post(hs[i], wtl, wth, bt, ne))
    prob = jnp.concatenate([p.reshape(-1) for p, _ in outs])
    attr_t = jnp.concatenate([t for _, t in outs], axis=1)
    return (prob, attr_t.T)


# speedup vs baseline: 2.5325x; 1.0412x over previous
import functools

import jax
import jax.numpy as jnp
from jax import lax
from jax.experimental import pallas as pl
from jax.experimental.pallas import tpu as pltpu
from jax.experimental.pallas import tpu_sc as plsc

N = 10000
E = 320000
D = 128
H2 = 64
EFD = 16

NC = 2
NS = 16
NW = NC * NS
EP = 327680
SLICES = 2
EPS = EP // SLICES
EPW = EPS // NW
CHUNK = 128
NCHUNK = EPW // CHUNK
NBUF = 2


def _pre_body(z_ref, wr_ref, wc_ref, bias_ref, a_ref, b_ref):
    zb = z_ref[...]
    s = jnp.sum(zb * zb, axis=1, keepdims=True)
    zn = zb / jnp.sqrt(s)
    a_ref[...] = (
        jnp.dot(zn, wr_ref[...], preferred_element_type=jnp.float32)
        + bias_ref[...]
    )
    b_ref[...] = jnp.dot(zn, wc_ref[...], preferred_element_type=jnp.float32)


def _tc_pre(z, wr, wc, bias):
    blk = 1000
    grid = N // blk
    return pl.pallas_call(
        _pre_body,
        grid=(grid,),
        in_specs=[
            pl.BlockSpec((blk, D), lambda i: (i, 0)),
            pl.BlockSpec((D, D), lambda i: (0, 0)),
            pl.BlockSpec((D, D), lambda i: (0, 0)),
            pl.BlockSpec((1, D), lambda i: (0, 0)),
        ],
        out_specs=[
            pl.BlockSpec((blk, D), lambda i: (i, 0)),
            pl.BlockSpec((blk, D), lambda i: (i, 0)),
        ],
        out_shape=[
            jax.ShapeDtypeStruct((N, D), jnp.float32),
            jax.ShapeDtypeStruct((N, D), jnp.float32),
        ],
    )(z, wr, wc, bias)


def _sc_body(a_hbm, b_hbm, row_hbm, col_hbm, out_hbm,
             ridx, cidx, ba0, ba1, bb0, bb1, hb0, hb1,
             sg0, sg1, sw0, sw1):
    wid = lax.axis_index("s") * NC + lax.axis_index("c")
    ba = (ba0, ba1)
    bb = (bb0, bb1)
    hb = (hb0, hb1)
    sg = (sg0, sg1)
    sw = (sw0, sw1)
    ebase = wid * EPW

    pltpu.sync_copy(row_hbm.at[pl.ds(wid * NCHUNK, NCHUNK)], ridx)
    pltpu.sync_copy(col_hbm.at[pl.ds(wid * NCHUNK, NCHUNK)], cidx)

    pltpu.async_copy(a_hbm.at[ridx.at[0]], ba[0], sg[0])
    pltpu.async_copy(b_hbm.at[cidx.at[0]], bb[0], sg[0])

    def outer(t, carry):
        for b in range(NBUF):
            c = t * NBUF + b
            s = b
            sn = (b + 1) % NBUF
            cn = lax.rem(c + 1, NCHUNK)
            pltpu.async_copy(a_hbm.at[ridx.at[cn]], ba[sn], sg[sn])
            pltpu.async_copy(b_hbm.at[cidx.at[cn]], bb[sn], sg[sn])
            pltpu.make_async_copy(a_hbm.at[pl.ds(0, CHUNK)], ba[s], sg[s]).wait()
            pltpu.make_async_copy(b_hbm.at[pl.ds(0, CHUNK)], bb[s], sg[s]).wait()

            @pl.when(c >= NBUF)
            def _():
                pltpu.make_async_copy(
                    hb[s], out_hbm.at[pl.ds(0, CHUNK)], sw[s]).wait()

            @plsc.parallel_loop(0, CHUNK, unroll=4)
            def _(r):
                for k in range(D // 16):
                    va = ba[s][r, pl.ds(k * 16, 16)]
                    vb = bb[s][r, pl.ds(k * 16, 16)]
                    hb[s][r, pl.ds(k * 16, 16)] = jnp.maximum(va + vb, 0.0)

            pltpu.async_copy(
                hb[s], out_hbm.at[pl.ds(ebase + c * CHUNK, CHUNK)], sw[s])
        return carry

    lax.fori_loop(0, NCHUNK // NBUF, outer, 0)

    pltpu.make_async_copy(a_hbm.at[pl.ds(0, CHUNK)], ba[0], sg[0]).wait()
    pltpu.make_async_copy(b_hbm.at[pl.ds(0, CHUNK)], bb[0], sg[0]).wait()
    for s in range(NBUF):
        pltpu.make_async_copy(hb[s], out_hbm.at[pl.ds(0, CHUNK)], sw[s]).wait()


def _sc_gather_relu(a, b, row2d, col2d):
    mesh = plsc.VectorSubcoreMesh(core_axis_name="c", subcore_axis_name="s")
    return pl.kernel(
        _sc_body,
        out_type=jax.ShapeDtypeStruct((EPS, D), jnp.float32),
        mesh=mesh,
        scratch_types=[
            pltpu.VMEM((NCHUNK, 128), jnp.int32),
            pltpu.VMEM((NCHUNK, 128), jnp.int32),
            pltpu.VMEM((CHUNK, D), jnp.float32),
            pltpu.VMEM((CHUNK, D), jnp.float32),
            pltpu.VMEM((CHUNK, D), jnp.float32),
            pltpu.VMEM((CHUNK, D), jnp.float32),
            pltpu.VMEM((CHUNK, D), jnp.float32),
            pltpu.VMEM((CHUNK, D), jnp.float32),
            pltpu.SemaphoreType.DMA,
            pltpu.SemaphoreType.DMA,
            pltpu.SemaphoreType.DMA,
            pltpu.SemaphoreType.DMA,
        ],
    )(a, b, row2d, col2d)


def _post_body(h_ref, wt_ref, bt_ref, prob_ref, attr_ref):
    h = h_ref[...]
    y_t = jax.lax.dot_general(
        wt_ref[...], h, (((1,), (1,)), ((), ())),
        preferred_element_type=jnp.float32,
    ) + bt_ref[...]
    prob_ref[...] = jax.nn.sigmoid(y_t[0:1, :])
    att = y_t[1:, :]
    att = att - jnp.max(att, axis=0, keepdims=True)
    ex = jnp.exp(att)
    attr_ref[...] = ex / jnp.sum(ex, axis=0, keepdims=True)


def _tc_post(h, wt, bt, ne):
    blk = 2560
    grid = ne // blk
    return pl.pallas_call(
        _post_body,
        grid=(grid,),
        in_specs=[
            pl.BlockSpec((blk, D), lambda i: (i, 0)),
            pl.BlockSpec((EFD + 1, D), lambda i: (0, 0)),
            pl.BlockSpec((EFD + 1, 1), lambda i: (0, 0)),
        ],
        out_specs=[
            pl.BlockSpec((1, blk), lambda i: (0, i)),
            pl.BlockSpec((EFD, blk), lambda i: (0, i)),
        ],
        out_shape=[
            jax.ShapeDtypeStruct((1, ne), jnp.float32),
            jax.ShapeDtypeStruct((EFD, ne), jnp.float32),
        ],
    )(h, wt, bt)


def kernel(z, out_edge_index, batch, mlp_w1, mlp_b1, mlp_w2, mlp_b2,
           efc1_w, efc1_b, efc2_w, efc2_b):
    wr = jnp.concatenate([mlp_w1[:D], efc1_w[:D]], axis=1)
    wc = jnp.concatenate([mlp_w1[D:], efc1_w[D:]], axis=1)
    bias = jnp.concatenate([mlp_b1, efc1_b]).reshape(1, 2 * H2)

    a, b = _tc_pre(z, wr, wc, bias)

    row = out_edge_index[0]
    col = out_edge_index[1]
    pad = EP - E
    row2d = jnp.concatenate([row, jnp.zeros((pad,), jnp.int32)]).reshape(-1, 128)
    col2d = jnp.concatenate([col, jnp.zeros((pad,), jnp.int32)]).reshape(-1, 128)

    hs = [
        _sc_gather_relu(a, b,
                        row2d[i * (EPS // 128):(i + 1) * (EPS // 128)],
                        col2d[i * (EPS // 128):(i + 1) * (EPS // 128)])
        for i in range(SLICES)
    ]

    wt = jnp.concatenate([
        jnp.concatenate([mlp_w2.reshape(1, H2),
                         jnp.zeros((1, H2), jnp.float32)], axis=1),
        jnp.concatenate([jnp.zeros((EFD, H2), jnp.float32),
                         efc2_w.T], axis=1),
    ], axis=0)
    bt = jnp.concatenate([mlp_b2, efc2_b]).reshape(EFD + 1, 1)

    outs = []
    for i in range(SLICES):
        ne = min(E - i * EPS, EPS)
        outs.append(_tc_post(hs[i], wt, bt, ne))
    prob = jnp.concatenate([p.reshape(-1) for p, _ in outs])
    attr_t = jnp.concatenate([t for _, t in outs], axis=1)
    return (prob, attr_t.T)
